# baseline (speedup 1.0000x reference)
import jax
import jax.numpy as jnp
from jax.experimental import pallas as pl
from jax.experimental.pallas import tpu as pltpu

LB_SMOOTH_ = 0.1
IGNORE_INDEX_ = 255
SUB = 16


def _ce_kernel(xa_ref, xb_ref, xc_ref, lab_ref, loss_ref, cnt_ref):
    hh = xa_ref.shape[2]
    w = xa_ref.shape[3]
    ca = xa_ref.shape[1]
    cb = xb_ref.shape[1]
    num_classes = ca + cb + xc_ref.shape[1]

    lb_pos = 1.0 - LB_SMOOTH_
    lb_neg = LB_SMOOTH_ / num_classes
    k_const = lb_pos + (num_classes - 1) * lb_neg

    def cls_ref(c):
        if c < ca:
            return xa_ref, c
        if c < ca + cb:
            return xb_ref, c - ca
        return xc_ref, c - ca - cb

    def tile_loss(row):
        lab = lab_ref[0, pl.ds(row, SUB), :]
        ignore = lab == IGNORE_INDEX_
        s = jnp.zeros((SUB, w), jnp.float32)
        wsum = jnp.zeros((SUB, w), jnp.float32)
        for c in range(num_classes):
            ref, cc = cls_ref(c)
            xcv = ref[0, cc, pl.ds(row, SUB), :]
            s = s + jnp.exp(xcv)
            wc = jnp.where(lab == c, lb_pos, lb_neg)
            wsum = wsum + wc * xcv
        loss = k_const * jnp.log(s) - wsum
        return jnp.where(ignore, 0.0, loss)

    def body(r, acc):
        return acc + tile_loss(r * SUB)

    loss_acc = jax.lax.fori_loop(
        0, hh // SUB, body, jnp.zeros((SUB, w), jnp.float32), unroll=8
    )
    cnt_all = jnp.where(lab_ref[0] == IGNORE_INDEX_, 0.0, 1.0)

    loss_ref[...] = jnp.sum(loss_acc).reshape(1, 1, 1)
    cnt_ref[...] = jnp.sum(cnt_all).reshape(1, 1, 1)


def kernel(logits, label):
    n, c, hh, w = logits.shape
    label = label.astype(jnp.int32)
    x = logits.astype(jnp.float32)

    loss_sums, cnts = pl.pallas_call(
        _ce_kernel,
        grid=(n,),
        in_specs=[
            pl.BlockSpec((1, 9, hh, w), lambda i: (i, 0, 0, 0)),
            pl.BlockSpec((1, 9, hh, w), lambda i: (i, 1, 0, 0)),
            pl.BlockSpec((1, 1, hh, w), lambda i: (i, 18, 0, 0)),
            pl.BlockSpec((1, hh, w), lambda i: (i, 0, 0)),
        ],
        out_specs=[
            pl.BlockSpec((1, 1, 1), lambda i: (i, 0, 0)),
            pl.BlockSpec((1, 1, 1), lambda i: (i, 0, 0)),
        ],
        out_shape=[
            jax.ShapeDtypeStruct((n, 1, 1), jnp.float32),
            jax.ShapeDtypeStruct((n, 1, 1), jnp.float32),
        ],
        compiler_params=pltpu.CompilerParams(
            dimension_semantics=("arbitrary",),
        ),
    )(x, x, x, label)

    return jnp.sum(loss_sums) / jnp.sum(cnts)


# speedup vs baseline: 1.0040x; 1.0040x over previous
import jax
import jax.numpy as jnp
from jax.experimental import pallas as pl
from jax.experimental.pallas import tpu as pltpu

LB_SMOOTH_ = 0.1
IGNORE_INDEX_ = 255
SUB = 16


def _ce_kernel(x_ref, lab_ref, loss_ref, cnt_ref):
    num_classes = x_ref.shape[1]
    hh = x_ref.shape[2]
    w = x_ref.shape[3]

    lb_pos = 1.0 - LB_SMOOTH_
    lb_neg = LB_SMOOTH_ / num_classes
    k_const = lb_pos + (num_classes - 1) * lb_neg

    def tile_loss(row):
        lab = lab_ref[0, pl.ds(row, SUB), :]
        ignore = lab == IGNORE_INDEX_
        s = jnp.zeros((SUB, w), jnp.float32)
        wsum = jnp.zeros((SUB, w), jnp.float32)
        for c in range(num_classes):
            xc = x_ref[0, c, pl.ds(row, SUB), :]
            s = s + jnp.exp(xc)
            wc = jnp.where(lab == c, lb_pos, lb_neg)
            wsum = wsum + wc * xc
        loss = k_const * jnp.log(s) - wsum
        return jnp.where(ignore, 0.0, loss)

    def body(r, acc):
        return acc + tile_loss(r * SUB)

    loss_acc = jax.lax.fori_loop(
        0, hh // SUB, body, jnp.zeros((SUB, w), jnp.float32), unroll=8
    )
    cnt_all = jnp.where(lab_ref[0] == IGNORE_INDEX_, 0.0, 1.0)

    loss_ref[...] = jnp.sum(loss_acc).reshape(1, 1, 1)
    cnt_ref[...] = jnp.sum(cnt_all).reshape(1, 1, 1)


def kernel(logits, label):
    n, c, hh, w = logits.shape
    label = label.astype(jnp.int32)

    loss_sums, cnts = pl.pallas_call(
        _ce_kernel,
        grid=(n,),
        in_specs=[
            pl.BlockSpec((1, c, hh, w), lambda i: (i, 0, 0, 0)),
            pl.BlockSpec((1, hh, w), lambda i: (i, 0, 0)),
        ],
        out_specs=[
            pl.BlockSpec((1, 1, 1), lambda i: (i, 0, 0)),
            pl.BlockSpec((1, 1, 1), lambda i: (i, 0, 0)),
        ],
        out_shape=[
            jax.ShapeDtypeStruct((n, 1, 1), jnp.float32),
            jax.ShapeDtypeStruct((n, 1, 1), jnp.float32),
        ],
        compiler_params=pltpu.CompilerParams(
            dimension_semantics=("arbitrary",),
        ),
    )(logits.astype(jnp.float32), label)

    return jnp.sum(loss_sums) / jnp.sum(cnts)
